# baseline (speedup 1.0000x reference)
import functools

import jax
import jax.numpy as jnp
from jax import lax
from jax.experimental import pallas as pl
from jax.experimental.pallas import tpu as pltpu
from jax.experimental.pallas import tpu_sc as plsc

NUM_CLASSES = 100000
FEAT_DIM = 64
BATCH = 16384

_INFO = plsc.get_sparse_core_info()
_NC = _INFO.num_cores
_NS = _INFO.num_subcores
_NW = _NC * _NS
_L = _INFO.num_lanes

_B_PER_W = BATCH // _NW
_CHUNK = 128
_NCHUNK = _B_PER_W // _CHUNK


def _body(x_hbm, lab_hbm, cen_hbm, out_hbm, idx_v, x_v, c_v, acc_v,
          s0, s1, s2, s3, xsem):
    wid = lax.axis_index("s") * _NC + lax.axis_index("c")
    base = wid * _NCHUNK

    pltpu.sync_copy(lab_hbm.at[pl.ds(base, _NCHUNK)], idx_v)
    xcopy = pltpu.async_copy(x_hbm.at[pl.ds(base, _NCHUNK)], x_v, xsem)

    sems = (s0, s1, s2, s3)
    gathers = [
        pltpu.async_copy(cen_hbm.at[idx_v.at[j]], c_v.at[j], sems[j])
        for j in range(_NCHUNK)
    ]
    xcopy.wait()

    def row_body(r, accs):
        a0, a1, a2, a3 = accs
        d0 = x_v[j, r, pl.ds(0, _L)] - c_v[j, r, pl.ds(0, _L)]
        d1 = x_v[j, r, pl.ds(_L, _L)] - c_v[j, r, pl.ds(_L, _L)]
        d2 = x_v[j, r, pl.ds(2 * _L, _L)] - c_v[j, r, pl.ds(2 * _L, _L)]
        d3 = x_v[j, r, pl.ds(3 * _L, _L)] - c_v[j, r, pl.ds(3 * _L, _L)]
        return (a0 + d0 * d0, a1 + d1 * d1, a2 + d2 * d2, a3 + d3 * d3)

    zero = jnp.zeros((_L,), jnp.float32)
    accs = (zero, zero, zero, zero)
    for j in range(_NCHUNK):
        gathers[j].wait()
        accs = lax.fori_loop(0, _CHUNK, row_body, accs, unroll=2)

    acc_v[...] = accs[0] + accs[1] + accs[2] + accs[3]
    pltpu.sync_copy(acc_v, out_hbm.at[wid])


@jax.jit
def _center_loss(x, labels, centers):
    x3 = x.reshape(_NW * _NCHUNK, _CHUNK, FEAT_DIM)
    lab = labels.astype(jnp.int32).reshape(_NW * _NCHUNK, _CHUNK)
    run = functools.partial(
        pl.kernel,
        out_type=jax.ShapeDtypeStruct((_NW, _L), jnp.float32),
        mesh=plsc.VectorSubcoreMesh(core_axis_name="c", subcore_axis_name="s"),
        compiler_params=pltpu.CompilerParams(use_tc_tiling_on_sc=False),
        scratch_types=[
            pltpu.VMEM((_NCHUNK, _CHUNK), jnp.int32),
            pltpu.VMEM((_NCHUNK, _CHUNK, FEAT_DIM), jnp.float32),
            pltpu.VMEM((_NCHUNK, _CHUNK, FEAT_DIM), jnp.float32),
            pltpu.VMEM((_L,), jnp.float32),
            pltpu.SemaphoreType.DMA,
            pltpu.SemaphoreType.DMA,
            pltpu.SemaphoreType.DMA,
            pltpu.SemaphoreType.DMA,
            pltpu.SemaphoreType.DMA,
        ],
    )(_body)
    partials = run(x3, lab, centers)
    return jnp.sum(partials) / 2.0 / BATCH


def kernel(x, labels, centers):
    return _center_loss(x, labels, centers)
